# baseline (device time: 18062 ns/iter reference)
import jax
import jax.numpy as jnp
from jax import lax
from jax.experimental import pallas as pl
from jax.experimental.pallas import tpu as pltpu

N_DEV = 4
SEG = 2
COMM_DTYPE = jnp.bfloat16


def kernel(x, w_mat):
    m_full, _ = x.shape
    _, n = w_mat.shape
    m_chunk = m_full // N_DEV
    nh = n // 2
    segw = nh // SEG

    def body(x_ref, w_ref, out_ref, p_ref, xb_ref, wb_ref,
             send_r, recv_r, send_l, recv_l,
             ssem_r, rsem_r, ssem_l, rsem_l):
        my = lax.axis_index("i")
        left = lax.rem(my + N_DEV - 1, N_DEV)
        right = lax.rem(my + 1, N_DEV)

        def rows(c):
            return pl.ds(lax.rem(c + 2 * N_DEV, N_DEV) * m_chunk, m_chunk)

        def colA(k):
            return slice(k * segw, (k + 1) * segw)

        def colB(k):
            return slice(nh + k * segw, nh + (k + 1) * segw)

        def rdma(buf_s, buf_r, ssem, rsem, s, k, dev):
            return pltpu.make_async_remote_copy(
                src_ref=buf_s.at[s, k], dst_ref=buf_r.at[s, k],
                send_sem=ssem.at[s, k], recv_sem=rsem.at[s, k],
                device_id=(dev,), device_id_type=pl.DeviceIdType.MESH,
            )

        barrier_sem = pltpu.get_barrier_semaphore()
        for nbr in (left, right):
            pl.semaphore_signal(
                barrier_sem, inc=1,
                device_id=(nbr,), device_id_type=pl.DeviceIdType.MESH,
            )
        pl.semaphore_wait(barrier_sem, 2)

        xb_ref[:, :] = x_ref[:, :].astype(COMM_DTYPE)
        wb_ref[:, :] = w_ref[:, :].astype(COMM_DTYPE)

        descs_r = {}
        descs_l = {}

        for k in range(SEG):
            send_r[0, k, :, :] = jnp.dot(
                xb_ref[rows(my - 1), :], wb_ref[:, colA(k)],
                preferred_element_type=jnp.float32,
            ).astype(COMM_DTYPE)
            d = rdma(send_r, recv_r, ssem_r, rsem_r, 0, k, right)
            d.start()
            descs_r[0, k] = d
            send_l[0, k, :, :] = jnp.dot(
                xb_ref[rows(my + 1), :], wb_ref[:, colB(k)],
                preferred_element_type=jnp.float32,
            ).astype(COMM_DTYPE)
            d = rdma(send_l, recv_l, ssem_l, rsem_l, 0, k, left)
            d.start()
            descs_l[0, k] = d

        p_ref[rows(my + 2), :] = jnp.dot(
            xb_ref[rows(my + 2), :], wb_ref[:, :],
            preferred_element_type=jnp.float32,
        )

        for k in range(SEG):
            descs_r[0, k].wait_recv()
            send_r[1, k, :, :] = (
                p_ref[rows(my + 2), colA(k)]
                + recv_r[0, k].astype(jnp.float32)
            ).astype(COMM_DTYPE)
            d = rdma(send_r, recv_r, ssem_r, rsem_r, 1, k, right)
            d.start()
            descs_r[1, k] = d
            descs_l[0, k].wait_recv()
            send_l[1, k, :, :] = (
                p_ref[rows(my + 2), colB(k)]
                + recv_l[0, k].astype(jnp.float32)
            ).astype(COMM_DTYPE)
            d = rdma(send_l, recv_l, ssem_l, rsem_l, 1, k, left)
            d.start()
            descs_l[1, k] = d

        p_ref[rows(my + 1), :nh] = jnp.dot(
            xb_ref[rows(my + 1), :], wb_ref[:, :nh],
            preferred_element_type=jnp.float32,
        )
        p_ref[rows(my - 1), nh:] = jnp.dot(
            xb_ref[rows(my - 1), :], wb_ref[:, nh:],
            preferred_element_type=jnp.float32,
        )
        p_ref[rows(my), :] = jnp.dot(
            xb_ref[rows(my), :], wb_ref[:, :],
            preferred_element_type=jnp.float32,
        )

        for k in range(SEG):
            descs_r[1, k].wait_recv()
            send_r[2, k, :, :] = (
                p_ref[rows(my + 1), colA(k)]
                + recv_r[1, k].astype(jnp.float32)
            ).astype(COMM_DTYPE)
            d = rdma(send_r, recv_r, ssem_r, rsem_r, 2, k, right)
            d.start()
            descs_r[2, k] = d
            descs_l[1, k].wait_recv()
            send_l[2, k, :, :] = (
                p_ref[rows(my - 1), colB(k)]
                + recv_l[1, k].astype(jnp.float32)
            ).astype(COMM_DTYPE)
            d = rdma(send_l, recv_l, ssem_l, rsem_l, 2, k, left)
            d.start()
            descs_l[2, k] = d

        for k in range(SEG):
            descs_r[2, k].wait_recv()
            out_ref[:, colA(k)] = (
                p_ref[rows(my), colA(k)] + recv_r[2, k].astype(jnp.float32)
            )
            descs_l[2, k].wait_recv()
            out_ref[:, colB(k)] = (
                p_ref[rows(my), colB(k)] + recv_l[2, k].astype(jnp.float32)
            )

        for d in list(descs_r.values()) + list(descs_l.values()):
            d.wait_send()

    comm_shape = (N_DEV - 1, SEG, m_chunk, segw)
    sem_shape = (N_DEV - 1, SEG)
    return pl.pallas_call(
        body,
        out_shape=jax.ShapeDtypeStruct((m_chunk, n), jnp.float32),
        in_specs=[
            pl.BlockSpec(memory_space=pltpu.VMEM),
            pl.BlockSpec(memory_space=pltpu.VMEM),
        ],
        out_specs=pl.BlockSpec(memory_space=pltpu.VMEM),
        scratch_shapes=[
            pltpu.VMEM((m_full, n), jnp.float32),
            pltpu.VMEM(x.shape, COMM_DTYPE),
            pltpu.VMEM(w_mat.shape, COMM_DTYPE),
            pltpu.VMEM(comm_shape, COMM_DTYPE),
            pltpu.VMEM(comm_shape, COMM_DTYPE),
            pltpu.VMEM(comm_shape, COMM_DTYPE),
            pltpu.VMEM(comm_shape, COMM_DTYPE),
            pltpu.SemaphoreType.DMA(sem_shape),
            pltpu.SemaphoreType.DMA(sem_shape),
            pltpu.SemaphoreType.DMA(sem_shape),
            pltpu.SemaphoreType.DMA(sem_shape),
        ],
        compiler_params=pltpu.CompilerParams(collective_id=0),
    )(x, w_mat)


# device time: 17395 ns/iter; 1.0383x vs baseline; 1.0383x over previous
import jax
import jax.numpy as jnp
from jax import lax
from jax.experimental import pallas as pl
from jax.experimental.pallas import tpu as pltpu

N_DEV = 4
SEG = 2
COMM_DTYPE = jnp.bfloat16

RAW_A, RAW_B, DIR_A, DIR_B, MRG_A, MRG_B = range(6)


def kernel(x, w_mat):
    m_full, _ = x.shape
    _, n = w_mat.shape
    m_chunk = m_full // N_DEV
    nh = n // 2
    segw = nh // SEG

    def body(x_ref, w_ref, out_ref, p_ref, sbuf, rbuf, ssem, rsem):
        my = lax.axis_index("i")
        left = lax.rem(my + N_DEV - 1, N_DEV)
        right = lax.rem(my + 1, N_DEV)

        def rows(c):
            return pl.ds(lax.rem(c + 2 * N_DEV, N_DEV) * m_chunk, m_chunk)

        def colA(k):
            return slice(k * segw, (k + 1) * segw)

        def colB(k):
            return slice(nh + k * segw, nh + (k + 1) * segw)

        def send(flow, k, dev):
            d = pltpu.make_async_remote_copy(
                src_ref=sbuf.at[flow, k], dst_ref=rbuf.at[flow, k],
                send_sem=ssem.at[flow, k], recv_sem=rsem.at[flow, k],
                device_id=(dev,), device_id_type=pl.DeviceIdType.MESH,
            )
            d.start()
            return d

        barrier_sem = pltpu.get_barrier_semaphore()
        for nbr in (left, right):
            pl.semaphore_signal(
                barrier_sem, inc=1,
                device_id=(nbr,), device_id_type=pl.DeviceIdType.MESH,
            )
        pl.semaphore_wait(barrier_sem, 2)

        descs = {}

        p_ref[rows(my + 2), :] = jnp.dot(
            x_ref[rows(my + 2), :], w_ref[:, :],
            preferred_element_type=jnp.float32,
        )
        for k in range(SEG):
            sbuf[RAW_A, k, :, :] = (
                p_ref[rows(my + 2), colA(k)].astype(COMM_DTYPE)
            )
            descs[RAW_A, k] = send(RAW_A, k, left)
            sbuf[RAW_B, k, :, :] = (
                p_ref[rows(my + 2), colB(k)].astype(COMM_DTYPE)
            )
            descs[RAW_B, k] = send(RAW_B, k, right)

        p_ref[rows(my + 1), :] = jnp.dot(
            x_ref[rows(my + 1), :], w_ref[:, :],
            preferred_element_type=jnp.float32,
        )
        for k in range(SEG):
            sbuf[DIR_A, k, :, :] = (
                p_ref[rows(my + 1), colA(k)].astype(COMM_DTYPE)
            )
            descs[DIR_A, k] = send(DIR_A, k, right)
        p_ref[rows(my - 1), :] = jnp.dot(
            x_ref[rows(my - 1), :], w_ref[:, :],
            preferred_element_type=jnp.float32,
        )
        for k in range(SEG):
            sbuf[DIR_B, k, :, :] = (
                p_ref[rows(my - 1), colB(k)].astype(COMM_DTYPE)
            )
            descs[DIR_B, k] = send(DIR_B, k, left)

        for k in range(SEG):
            descs[RAW_A, k].wait_recv()
            sbuf[MRG_A, k, :, :] = (
                p_ref[rows(my - 1), colA(k)]
                + rbuf[RAW_A, k].astype(jnp.float32)
            ).astype(COMM_DTYPE)
            descs[MRG_A, k] = send(MRG_A, k, left)
            descs[RAW_B, k].wait_recv()
            sbuf[MRG_B, k, :, :] = (
                p_ref[rows(my + 1), colB(k)]
                + rbuf[RAW_B, k].astype(jnp.float32)
            ).astype(COMM_DTYPE)
            descs[MRG_B, k] = send(MRG_B, k, right)

        p_ref[rows(my), :] = jnp.dot(
            x_ref[rows(my), :], w_ref[:, :],
            preferred_element_type=jnp.float32,
        )

        for k in range(SEG):
            descs[DIR_A, k].wait_recv()
            descs[MRG_A, k].wait_recv()
            out_ref[:, colA(k)] = (
                p_ref[rows(my), colA(k)]
                + rbuf[DIR_A, k].astype(jnp.float32)
                + rbuf[MRG_A, k].astype(jnp.float32)
            )
            descs[DIR_B, k].wait_recv()
            descs[MRG_B, k].wait_recv()
            out_ref[:, colB(k)] = (
                p_ref[rows(my), colB(k)]
                + rbuf[DIR_B, k].astype(jnp.float32)
                + rbuf[MRG_B, k].astype(jnp.float32)
            )

        for d in descs.values():
            d.wait_send()

    comm_shape = (6, SEG, m_chunk, segw)
    sem_shape = (6, SEG)
    return pl.pallas_call(
        body,
        out_shape=jax.ShapeDtypeStruct((m_chunk, n), jnp.float32),
        in_specs=[
            pl.BlockSpec(memory_space=pltpu.VMEM),
            pl.BlockSpec(memory_space=pltpu.VMEM),
        ],
        out_specs=pl.BlockSpec(memory_space=pltpu.VMEM),
        scratch_shapes=[
            pltpu.VMEM((m_full, n), jnp.float32),
            pltpu.VMEM(comm_shape, COMM_DTYPE),
            pltpu.VMEM(comm_shape, COMM_DTYPE),
            pltpu.SemaphoreType.DMA(sem_shape),
            pltpu.SemaphoreType.DMA(sem_shape),
        ],
        compiler_params=pltpu.CompilerParams(collective_id=0),
    )(x, w_mat)


# device time: 17353 ns/iter; 1.0409x vs baseline; 1.0024x over previous
import jax
import jax.numpy as jnp
from jax import lax
from jax.experimental import pallas as pl
from jax.experimental.pallas import tpu as pltpu

N_DEV = 4
SEG = 1
COMM_DTYPE = jnp.bfloat16

RAW_A, RAW_B, DIR_A, DIR_B, MRG_A, MRG_B = range(6)


def kernel(x, w_mat):
    m_full, _ = x.shape
    _, n = w_mat.shape
    m_chunk = m_full // N_DEV
    nh = n // 2
    segw = nh // SEG

    def body(x_ref, w_ref, out_ref, p_ref, sbuf, rbuf, ssem, rsem):
        my = lax.axis_index("i")
        left = lax.rem(my + N_DEV - 1, N_DEV)
        right = lax.rem(my + 1, N_DEV)

        def rows(c):
            return pl.ds(lax.rem(c + 2 * N_DEV, N_DEV) * m_chunk, m_chunk)

        def colA(k):
            return slice(k * segw, (k + 1) * segw)

        def colB(k):
            return slice(nh + k * segw, nh + (k + 1) * segw)

        def send(flow, k, dev):
            d = pltpu.make_async_remote_copy(
                src_ref=sbuf.at[flow, k], dst_ref=rbuf.at[flow, k],
                send_sem=ssem.at[flow, k], recv_sem=rsem.at[flow, k],
                device_id=(dev,), device_id_type=pl.DeviceIdType.MESH,
            )
            d.start()
            return d

        barrier_sem = pltpu.get_barrier_semaphore()
        for nbr in (left, right):
            pl.semaphore_signal(
                barrier_sem, inc=1,
                device_id=(nbr,), device_id_type=pl.DeviceIdType.MESH,
            )
        pl.semaphore_wait(barrier_sem, 2)

        descs = {}

        p_ref[rows(my + 2), :] = jnp.dot(
            x_ref[rows(my + 2), :], w_ref[:, :],
            preferred_element_type=jnp.float32,
        )
        for k in range(SEG):
            sbuf[RAW_A, k, :, :] = (
                p_ref[rows(my + 2), colA(k)].astype(COMM_DTYPE)
            )
            descs[RAW_A, k] = send(RAW_A, k, left)
            sbuf[RAW_B, k, :, :] = (
                p_ref[rows(my + 2), colB(k)].astype(COMM_DTYPE)
            )
            descs[RAW_B, k] = send(RAW_B, k, right)

        p_ref[rows(my + 1), :] = jnp.dot(
            x_ref[rows(my + 1), :], w_ref[:, :],
            preferred_element_type=jnp.float32,
        )
        for k in range(SEG):
            sbuf[DIR_A, k, :, :] = (
                p_ref[rows(my + 1), colA(k)].astype(COMM_DTYPE)
            )
            descs[DIR_A, k] = send(DIR_A, k, right)
        p_ref[rows(my - 1), :] = jnp.dot(
            x_ref[rows(my - 1), :], w_ref[:, :],
            preferred_element_type=jnp.float32,
        )
        for k in range(SEG):
            sbuf[DIR_B, k, :, :] = (
                p_ref[rows(my - 1), colB(k)].astype(COMM_DTYPE)
            )
            descs[DIR_B, k] = send(DIR_B, k, left)

        for k in range(SEG):
            descs[RAW_A, k].wait_recv()
            sbuf[MRG_A, k, :, :] = (
                p_ref[rows(my - 1), colA(k)]
                + rbuf[RAW_A, k].astype(jnp.float32)
            ).astype(COMM_DTYPE)
            descs[MRG_A, k] = send(MRG_A, k, left)
            descs[RAW_B, k].wait_recv()
            sbuf[MRG_B, k, :, :] = (
                p_ref[rows(my + 1), colB(k)]
                + rbuf[RAW_B, k].astype(jnp.float32)
            ).astype(COMM_DTYPE)
            descs[MRG_B, k] = send(MRG_B, k, right)

        p_ref[rows(my), :] = jnp.dot(
            x_ref[rows(my), :], w_ref[:, :],
            preferred_element_type=jnp.float32,
        )

        for k in range(SEG):
            descs[DIR_A, k].wait_recv()
            descs[MRG_A, k].wait_recv()
            out_ref[:, colA(k)] = (
                p_ref[rows(my), colA(k)]
                + rbuf[DIR_A, k].astype(jnp.float32)
                + rbuf[MRG_A, k].astype(jnp.float32)
            )
            descs[DIR_B, k].wait_recv()
            descs[MRG_B, k].wait_recv()
            out_ref[:, colB(k)] = (
                p_ref[rows(my), colB(k)]
                + rbuf[DIR_B, k].astype(jnp.float32)
                + rbuf[MRG_B, k].astype(jnp.float32)
            )

        for d in descs.values():
            d.wait_send()

    comm_shape = (6, SEG, m_chunk, segw)
    sem_shape = (6, SEG)
    return pl.pallas_call(
        body,
        out_shape=jax.ShapeDtypeStruct((m_chunk, n), jnp.float32),
        in_specs=[
            pl.BlockSpec(memory_space=pltpu.VMEM),
            pl.BlockSpec(memory_space=pltpu.VMEM),
        ],
        out_specs=pl.BlockSpec(memory_space=pltpu.VMEM),
        scratch_shapes=[
            pltpu.VMEM((m_full, n), jnp.float32),
            pltpu.VMEM(comm_shape, COMM_DTYPE),
            pltpu.VMEM(comm_shape, COMM_DTYPE),
            pltpu.SemaphoreType.DMA(sem_shape),
            pltpu.SemaphoreType.DMA(sem_shape),
        ],
        compiler_params=pltpu.CompilerParams(collective_id=0),
    )(x, w_mat)
